# baseline (speedup 1.0000x reference)
import functools

import jax
import jax.numpy as jnp
from jax.experimental import pallas as pl
from jax.experimental.pallas import tpu as pltpu

TOP_K = 8
N_EXPERTS = 64
N_GROUP = 8
GROUP_SIZE = N_EXPERTS // N_GROUP
TOPK_GROUP = 4
SCALE = 2.5

TILE = 512


def _gate_kernel(x_ref, w_ref, b_ref, idx_ref, wgt_ref):
    x = x_ref[...]
    w = w_ref[...]
    logits = jax.lax.dot_general(
        x, w, (((1,), (1,)), ((), ())),
        preferred_element_type=jnp.float32,
    )
    s = jax.nn.sigmoid(logits)
    sc = s + b_ref[...]

    t = x.shape[0]
    lane = jax.lax.broadcasted_iota(jnp.int32, (t, N_EXPERTS), 1)
    gid = lane // GROUP_SIZE
    neg = jnp.float32(-jnp.inf)

    top2 = []
    for g in range(N_GROUP):
        gm = jnp.where(gid == g, sc, neg)
        m1 = jnp.max(gm, axis=-1, keepdims=True)
        i1 = jnp.min(jnp.where(gm == m1, lane, N_EXPERTS), axis=-1,
                     keepdims=True)
        m2 = jnp.max(jnp.where(lane == i1, neg, gm), axis=-1, keepdims=True)
        top2.append(m1 + m2)

    keep_bcast = jnp.zeros_like(sc)
    for g in range(N_GROUP):
        rank = jnp.zeros_like(top2[g], dtype=jnp.int32)
        for h in range(N_GROUP):
            if h == g:
                continue
            beats = (top2[h] > top2[g]) | ((top2[h] == top2[g]) & (h < g))
            rank = rank + beats.astype(jnp.int32)
        keep_g = (rank < TOPK_GROUP).astype(jnp.float32)
        keep_bcast = jnp.where(gid == g, keep_g, keep_bcast)

    tmp = jnp.where(keep_bcast > 0, sc, neg)

    idx_cols, w_cols = [], []
    for _ in range(TOP_K):
        m = jnp.max(tmp, axis=-1, keepdims=True)
        i = jnp.min(jnp.where(tmp == m, lane, N_EXPERTS), axis=-1,
                    keepdims=True)
        onehot = lane == i
        w_cols.append(jnp.sum(jnp.where(onehot, s, 0.0), axis=-1,
                              keepdims=True))
        idx_cols.append(i)
        tmp = jnp.where(onehot, neg, tmp)

    idx = jnp.concatenate(idx_cols, axis=1)
    wgt = jnp.concatenate(w_cols, axis=1)
    denom = jnp.sum(wgt, axis=-1, keepdims=True) + 1e-20
    wgt = wgt * (SCALE / denom)

    idx_ref[...] = idx
    wgt_ref[...] = wgt


@functools.partial(jax.jit, static_argnames=())
def kernel(hidden_states, weight, e_score_correction_bias):
    bsz, seq_len, h = hidden_states.shape
    n = bsz * seq_len
    x = hidden_states.reshape(n, h).astype(jnp.float32)
    w = weight.astype(jnp.float32)
    b = e_score_correction_bias.astype(jnp.float32).reshape(1, N_EXPERTS)

    grid = (n // TILE,)
    idx, wgt = pl.pallas_call(
        _gate_kernel,
        grid=grid,
        in_specs=[
            pl.BlockSpec((TILE, h), lambda i: (i, 0)),
            pl.BlockSpec((N_EXPERTS, h), lambda i: (0, 0)),
            pl.BlockSpec((1, N_EXPERTS), lambda i: (0, 0)),
        ],
        out_specs=[
            pl.BlockSpec((TILE, TOP_K), lambda i: (i, 0)),
            pl.BlockSpec((TILE, TOP_K), lambda i: (i, 0)),
        ],
        out_shape=[
            jax.ShapeDtypeStruct((n, TOP_K), jnp.int32),
            jax.ShapeDtypeStruct((n, TOP_K), jnp.float32),
        ],
        compiler_params=pltpu.CompilerParams(
            dimension_semantics=("parallel",),
        ),
    )(x, w, b)
    return idx, wgt


# speedup vs baseline: 1.2567x; 1.2567x over previous
import functools

import jax
import jax.numpy as jnp
from jax.experimental import pallas as pl
from jax.experimental.pallas import tpu as pltpu

TOP_K = 8
N_EXPERTS = 64
N_GROUP = 8
GROUP_SIZE = N_EXPERTS // N_GROUP
TOPK_GROUP = 4
SCALE = 2.5

TILE = 512


def _gate_kernel(x_ref, w_ref, b_ref, idx_ref, wgt_ref):
    x = x_ref[...]
    w = w_ref[...]
    logits = jax.lax.dot_general(
        x, w, (((1,), (1,)), ((), ())),
        preferred_element_type=jnp.float32,
    )
    s = jax.nn.sigmoid(logits)
    sc = s + b_ref[...]

    t = x.shape[0]
    lane = jax.lax.broadcasted_iota(jnp.int32, (t, N_EXPERTS), 1)
    gid = lane // GROUP_SIZE
    lanem = lane % GROUP_SIZE
    neg = jnp.float32(-jnp.inf)

    a = sc
    b = jnp.full_like(sc, neg)
    for d in (1, 2, 4):
        ar = jnp.roll(a, -d, axis=1)
        br = jnp.roll(b, -d, axis=1)
        valid = lanem < (GROUP_SIZE - d)
        ar = jnp.where(valid, ar, neg)
        br = jnp.where(valid, br, neg)
        b = jnp.maximum(jnp.maximum(b, br), jnp.minimum(a, ar))
        a = jnp.maximum(a, ar)
    g = a + b
    for d in (1, 2, 4):
        g = jnp.where(lanem >= d, jnp.roll(g, d, axis=1), g)

    rank = jnp.zeros_like(lane)
    for k in range(1, N_GROUP):
        other = jnp.roll(g, -GROUP_SIZE * k, axis=1)
        og_lt = gid >= (N_GROUP - k)
        beats = (other > g) | ((other == g) & og_lt)
        rank = rank + beats.astype(jnp.int32)
    tmp = jnp.where(rank < TOPK_GROUP, sc, neg)

    idx_cols, w_cols = [], []
    for _ in range(TOP_K):
        m = jnp.max(tmp, axis=-1, keepdims=True)
        i = jnp.min(jnp.where(tmp == m, lane, N_EXPERTS), axis=-1,
                    keepdims=True)
        onehot = lane == i
        w_cols.append(jnp.sum(jnp.where(onehot, s, 0.0), axis=-1,
                              keepdims=True))
        idx_cols.append(i)
        tmp = jnp.where(onehot, neg, tmp)

    idx = jnp.concatenate(idx_cols, axis=1)
    wgt = jnp.concatenate(w_cols, axis=1)
    denom = jnp.sum(wgt, axis=-1, keepdims=True) + 1e-20
    wgt = wgt * (SCALE / denom)

    idx_ref[...] = idx
    wgt_ref[...] = wgt


@functools.partial(jax.jit, static_argnames=())
def kernel(hidden_states, weight, e_score_correction_bias):
    bsz, seq_len, h = hidden_states.shape
    n = bsz * seq_len
    x = hidden_states.reshape(n, h).astype(jnp.float32)
    w = weight.astype(jnp.float32)
    b = e_score_correction_bias.astype(jnp.float32).reshape(1, N_EXPERTS)

    grid = (n // TILE,)
    idx, wgt = pl.pallas_call(
        _gate_kernel,
        grid=grid,
        in_specs=[
            pl.BlockSpec((TILE, h), lambda i: (i, 0)),
            pl.BlockSpec((N_EXPERTS, h), lambda i: (0, 0)),
            pl.BlockSpec((1, N_EXPERTS), lambda i: (0, 0)),
        ],
        out_specs=[
            pl.BlockSpec((TILE, TOP_K), lambda i: (i, 0)),
            pl.BlockSpec((TILE, TOP_K), lambda i: (i, 0)),
        ],
        out_shape=[
            jax.ShapeDtypeStruct((n, TOP_K), jnp.int32),
            jax.ShapeDtypeStruct((n, TOP_K), jnp.float32),
        ],
        compiler_params=pltpu.CompilerParams(
            dimension_semantics=("parallel",),
        ),
    )(x, w, b)
    return idx, wgt


# speedup vs baseline: 1.5022x; 1.1953x over previous
import functools

import jax
import jax.numpy as jnp
from jax.experimental import pallas as pl
from jax.experimental.pallas import tpu as pltpu

TOP_K = 8
N_EXPERTS = 64
N_GROUP = 8
GROUP_SIZE = N_EXPERTS // N_GROUP
TOPK_GROUP = 4
SCALE = 2.5

TILE = 512


def _gate_kernel(x_ref, w_ref, b_ref, idx_ref, wgt_ref):
    x = x_ref[...]
    w = w_ref[...]
    logits = jax.lax.dot_general(
        x, w, (((1,), (1,)), ((), ())),
        preferred_element_type=jnp.float32,
    )
    s = jax.nn.sigmoid(logits)
    sc = s + b_ref[...]

    t = x.shape[0]
    lane = jax.lax.broadcasted_iota(jnp.int32, (t, N_EXPERTS), 1)
    lane_f = lane.astype(jnp.float32)
    gid = lane // GROUP_SIZE
    lanem = lane % GROUP_SIZE
    neg = jnp.float32(-jnp.inf)

    a = sc
    b = jnp.full_like(sc, neg)
    for d in (1, 2, 4):
        ar = jnp.roll(a, -d, axis=1)
        br = jnp.roll(b, -d, axis=1)
        valid = lanem < (GROUP_SIZE - d)
        ar = jnp.where(valid, ar, neg)
        br = jnp.where(valid, br, neg)
        b = jnp.maximum(jnp.maximum(b, br), jnp.minimum(a, ar))
        a = jnp.maximum(a, ar)
    g = a + b
    for d in (1, 2, 4):
        g = jnp.where(lanem >= d, jnp.roll(g, d, axis=1), g)

    rank = jnp.zeros_like(lane)
    for k in range(1, N_GROUP):
        other = jnp.roll(g, -GROUP_SIZE * k, axis=1)
        og_lt = gid >= (N_GROUP - k)
        beats = (other > g) | ((other == g) & og_lt)
        rank = rank + beats.astype(jnp.int32)
    tmp = jnp.where(rank < TOPK_GROUP, sc, neg)

    idx_cols, w_cols = [], []
    for _ in range(TOP_K):
        m = jnp.max(tmp, axis=-1, keepdims=True)
        i = jnp.min(jnp.where(tmp == m, lane_f, jnp.float32(N_EXPERTS)),
                    axis=-1, keepdims=True)
        onehot = lane_f == i
        w_cols.append(jnp.sum(jnp.where(onehot, s, 0.0), axis=-1,
                              keepdims=True))
        idx_cols.append(i)
        tmp = jnp.where(onehot, neg, tmp)

    idx = jnp.concatenate(idx_cols, axis=1).astype(jnp.int32)
    wgt = jnp.concatenate(w_cols, axis=1)
    denom = jnp.sum(wgt, axis=-1, keepdims=True) + 1e-20
    wgt = wgt * (SCALE / denom)

    idx_ref[...] = idx
    wgt_ref[...] = wgt


@functools.partial(jax.jit, static_argnames=())
def kernel(hidden_states, weight, e_score_correction_bias):
    bsz, seq_len, h = hidden_states.shape
    n = bsz * seq_len
    x = hidden_states.reshape(n, h).astype(jnp.float32)
    w = weight.astype(jnp.float32)
    b = e_score_correction_bias.astype(jnp.float32).reshape(1, N_EXPERTS)

    grid = (n // TILE,)
    idx, wgt = pl.pallas_call(
        _gate_kernel,
        grid=grid,
        in_specs=[
            pl.BlockSpec((TILE, h), lambda i: (i, 0)),
            pl.BlockSpec((N_EXPERTS, h), lambda i: (0, 0)),
            pl.BlockSpec((1, N_EXPERTS), lambda i: (0, 0)),
        ],
        out_specs=[
            pl.BlockSpec((TILE, TOP_K), lambda i: (i, 0)),
            pl.BlockSpec((TILE, TOP_K), lambda i: (i, 0)),
        ],
        out_shape=[
            jax.ShapeDtypeStruct((n, TOP_K), jnp.int32),
            jax.ShapeDtypeStruct((n, TOP_K), jnp.float32),
        ],
        compiler_params=pltpu.CompilerParams(
            dimension_semantics=("parallel",),
        ),
    )(x, w, b)
    return idx, wgt


# speedup vs baseline: 1.6078x; 1.0703x over previous
import functools

import jax
import jax.numpy as jnp
from jax.experimental import pallas as pl
from jax.experimental.pallas import tpu as pltpu

TOP_K = 8
N_EXPERTS = 64
N_GROUP = 8
GROUP_SIZE = N_EXPERTS // N_GROUP
TOPK_GROUP = 4
SCALE = 2.5

TILE = 1024


def _gate_kernel(x_ref, w_ref, b_ref, idx_ref, wgt_ref):
    x = x_ref[...]
    w = w_ref[...]
    logits = jax.lax.dot_general(
        x, w, (((1,), (1,)), ((), ())),
        preferred_element_type=jnp.float32,
    )
    s = jax.nn.sigmoid(logits)
    sc = s + b_ref[...]

    t = x.shape[0]
    lane = jax.lax.broadcasted_iota(jnp.int32, (t, N_EXPERTS), 1)
    lane_f = lane.astype(jnp.float32)
    gid = lane // GROUP_SIZE
    lanem = lane % GROUP_SIZE
    neg = jnp.float32(-jnp.inf)

    a = sc
    b = jnp.full_like(sc, neg)
    for d in (1, 2, 4):
        ar = jnp.roll(a, -d, axis=1)
        br = jnp.roll(b, -d, axis=1)
        valid = lanem < (GROUP_SIZE - d)
        ar = jnp.where(valid, ar, neg)
        br = jnp.where(valid, br, neg)
        b = jnp.maximum(jnp.maximum(b, br), jnp.minimum(a, ar))
        a = jnp.maximum(a, ar)
    g = a + b
    for d in (1, 2, 4):
        g = jnp.where(lanem >= d, jnp.roll(g, d, axis=1), g)

    rank = jnp.zeros_like(lane)
    for k in range(1, N_GROUP):
        other = jnp.roll(g, -GROUP_SIZE * k, axis=1)
        og_lt = gid >= (N_GROUP - k)
        beats = (other > g) | ((other == g) & og_lt)
        rank = rank + beats.astype(jnp.int32)
    tmp = jnp.where(rank < TOPK_GROUP, sc, neg)

    idx_cols, w_cols = [], []
    for _ in range(TOP_K):
        m = jnp.max(tmp, axis=-1, keepdims=True)
        i = jnp.min(jnp.where(tmp == m, lane_f, jnp.float32(N_EXPERTS)),
                    axis=-1, keepdims=True)
        onehot = lane_f == i
        w_cols.append(jnp.sum(jnp.where(onehot, s, 0.0), axis=-1,
                              keepdims=True))
        idx_cols.append(i)
        tmp = jnp.where(onehot, neg, tmp)

    idx = jnp.concatenate(idx_cols, axis=1).astype(jnp.int32)
    wgt = jnp.concatenate(w_cols, axis=1)
    denom = jnp.sum(wgt, axis=-1, keepdims=True) + 1e-20
    wgt = wgt * (SCALE / denom)

    idx_ref[...] = idx
    wgt_ref[...] = wgt


@functools.partial(jax.jit, static_argnames=())
def kernel(hidden_states, weight, e_score_correction_bias):
    bsz, seq_len, h = hidden_states.shape
    n = bsz * seq_len
    x = hidden_states.reshape(n, h).astype(jnp.float32)
    w = weight.astype(jnp.float32)
    b = e_score_correction_bias.astype(jnp.float32).reshape(1, N_EXPERTS)

    grid = (n // TILE,)
    idx, wgt = pl.pallas_call(
        _gate_kernel,
        grid=grid,
        in_specs=[
            pl.BlockSpec((TILE, h), lambda i: (i, 0)),
            pl.BlockSpec((N_EXPERTS, h), lambda i: (0, 0)),
            pl.BlockSpec((1, N_EXPERTS), lambda i: (0, 0)),
        ],
        out_specs=[
            pl.BlockSpec((TILE, TOP_K), lambda i: (i, 0)),
            pl.BlockSpec((TILE, TOP_K), lambda i: (i, 0)),
        ],
        out_shape=[
            jax.ShapeDtypeStruct((n, TOP_K), jnp.int32),
            jax.ShapeDtypeStruct((n, TOP_K), jnp.float32),
        ],
        compiler_params=pltpu.CompilerParams(
            dimension_semantics=("parallel",),
        ),
    )(x, w, b)
    return idx, wgt
